# baseline (speedup 1.0000x reference)
import functools

import jax
import jax.numpy as jnp
from jax import lax
from jax.experimental import pallas as pl
from jax.experimental.pallas import tpu as pltpu
from jax.experimental.pallas import tpu_sc as plsc

F32 = jnp.float32
I32 = jnp.int32

NC = 2
NS = 16
L = 16
NW = NC * NS

_MESH = dict(core_axis_name="c", subcore_axis_name="s")
_NO_LAYOUT = dict(
    compiler_params=pltpu.CompilerParams(needs_layout_passes=False)
)


def _wid():
    return lax.axis_index("s") * NC + lax.axis_index("c")


def _nodes_gather(vert_flat, nodes_idx):
    N = nodes_idx.shape[0]
    NPT = -(-N // NW)
    NPT = -(-NPT // L) * L

    @functools.partial(
        pl.kernel,
        mesh=plsc.VectorSubcoreMesh(**_MESH),
        **_NO_LAYOUT,
        out_type=jax.ShapeDtypeStruct((3 * N,), F32),
        scratch_types=[
            pltpu.VMEM((NPT,), I32),
            pltpu.VMEM((3 * NPT,), I32),
            pltpu.VMEM((3 * NPT,), F32),
            pltpu.VMEM((3 * NPT,), F32),
            pltpu.SemaphoreType.DMA,
        ],
    )
    def k(vert_hbm, idx_hbm, out_hbm, idx_v, idx3_v, rows_v, nt_v, sem):
        base = jnp.minimum(_wid() * NPT, N - NPT)
        pltpu.sync_copy(idx_hbm.at[pl.ds(base, NPT)], idx_v)
        iota = lax.iota(I32, L)
        for g in range(NPT // L):
            j3 = idx_v[pl.ds(g * L, L)] * 3
            p3 = (iota + g * L) * 3
            for c in range(3):
                plsc.store_scatter(idx3_v, [p3 + c], j3 + c)
        pltpu.async_copy(vert_hbm.at[idx3_v], rows_v, sem).wait()
        for g in range(NPT // L):
            lane3 = (iota + g * L) * 3
            for c in range(3):
                nt_v[pl.ds(c * NPT + g * L, L)] = plsc.load_gather(
                    rows_v, [lane3 + c]
                )
        for c in range(3):
            pltpu.sync_copy(
                nt_v.at[pl.ds(c * NPT, NPT)],
                out_hbm.at[pl.ds(c * N + base, NPT)],
            )

    return k(vert_flat, nodes_idx)


def _prep(rot_t, t_t, nodes_t):
    N = rot_t.shape[1]

    def body(rot_ref, t_ref, nt_ref, rb_ref, cs_ref):
        eps = jnp.float32(1e-8)
        rx = rot_ref[0:1, :]
        ry = rot_ref[1:2, :]
        rz = rot_ref[2:3, :]
        ax = rx + eps
        ay = ry + eps
        az = rz + eps
        angle = jnp.sqrt(ax * ax + ay * ay + az * az)
        inv = 1.0 / angle
        ux = rx * inv
        uy = ry * inv
        uz = rz * inv
        sn = jnp.sin(angle)
        one_c = 1.0 - jnp.cos(angle)
        xx = ux * ux
        yy = uy * uy
        zz = uz * uz
        xy = ux * uy
        xz = ux * uz
        yz = uy * uz
        r00 = 1.0 + one_c * (-zz - yy)
        r01 = sn * (-uz) + one_c * xy
        r02 = sn * uy + one_c * xz
        r10 = sn * uz + one_c * xy
        r11 = 1.0 + one_c * (-zz - xx)
        r12 = sn * (-ux) + one_c * yz
        r20 = sn * (-uy) + one_c * xz
        r21 = sn * ux + one_c * yz
        r22 = 1.0 + one_c * (-yy - xx)
        nx = nt_ref[0:1, :]
        ny = nt_ref[1:2, :]
        nz = nt_ref[2:3, :]
        tx = t_ref[0:1, :]
        ty = t_ref[1:2, :]
        tz = t_ref[2:3, :]
        bx = nx + tx - (r00 * nx + r01 * ny + r02 * nz)
        by = ny + ty - (r10 * nx + r11 * ny + r12 * nz)
        bz = nz + tz - (r20 * nx + r21 * ny + r22 * nz)
        for i, row in enumerate(
            [r00, r01, r02, r10, r11, r12, r20, r21, r22, bx, by, bz]
        ):
            rb_ref[i : i + 1, :] = row
        for i, row in enumerate([nx + tx, ny + ty, nz + tz, nx, ny, nz]):
            cs_ref[i : i + 1, :] = row

    return pl.pallas_call(
        body,
        out_shape=[
            jax.ShapeDtypeStruct((12, N), F32),
            jax.ShapeDtypeStruct((6, N), F32),
        ],
    )(rot_t, t_t, nodes_t)


def _warp_arap(vert_flat, inf_flat, w_flat, rb_flat, cs_flat, neigh_flat,
               V, K, N, M):
    VPT = 1568
    NCH = 7
    CH = VPT // NCH
    CG = CH // L
    NPT = -(-N // NW)
    NPT = -(-NPT // L) * L
    NGA = NPT // L

    @functools.partial(
        pl.kernel,
        mesh=plsc.VectorSubcoreMesh(**_MESH),
        **_NO_LAYOUT,
        out_type=[
            jax.ShapeDtypeStruct((V * 3,), F32),
            jax.ShapeDtypeStruct((NW * L,), F32),
        ],
        scratch_types=[
            pltpu.VMEM((12 * N,), F32),
            pltpu.VMEM((6 * N,), F32),
            pltpu.VMEM((NPT * M,), I32),
            pltpu.VMEM((CH * K,), I32),
            pltpu.VMEM((CH * K,), I32),
            pltpu.VMEM((CH * K,), F32),
            pltpu.VMEM((CH * K,), F32),
            pltpu.VMEM((VPT * 3,), F32),
            pltpu.VMEM((VPT * 3,), F32),
            pltpu.VMEM((L,), F32),
            pltpu.SemaphoreType.DMA,
            pltpu.SemaphoreType.DMA,
            pltpu.SemaphoreType.DMA,
            pltpu.SemaphoreType.DMA,
        ],
    )
    def k(vert_hbm, idx_hbm, w_hbm, rb_hbm, cs_hbm, neigh_hbm,
          out_hbm, part_hbm,
          rb_v, cs_v, neigh_v, idx0_v, idx1_v, w0_v, w1_v, vert_v, out_v,
          part_v, si0, si1, sw0, sw1):
        wid = _wid()
        base = jnp.minimum(wid * VPT, V - VPT)
        basen = jnp.minimum(wid * NPT, N - NPT)
        idxb = [idx0_v, idx1_v]
        wb = [w0_v, w1_v]
        sems = [(si0, sw0), (si1, sw1)]
        iota = lax.iota(I32, L)

        def fire(slot, ci):
            off = base * K + ci * (CH * K)
            hi = pltpu.async_copy(
                idx_hbm.at[pl.ds(off, CH * K)], idxb[slot], sems[slot][0]
            )
            hw = pltpu.async_copy(
                w_hbm.at[pl.ds(off, CH * K)], wb[slot], sems[slot][1]
            )
            return hi, hw

        handles = [None, None]
        handles[0] = fire(0, 0)
        pltpu.sync_copy(rb_hbm, rb_v)
        pltpu.sync_copy(cs_hbm, cs_v)
        pltpu.sync_copy(neigh_hbm.at[pl.ds(basen * M, NPT * M)], neigh_v)
        pltpu.sync_copy(vert_hbm.at[pl.ds(base * 3, VPT * 3)], vert_v)

        def arap_body(g, tot):
            off = g * L
            laneM = (iota + off) * M
            n_glob = basen + off + iota
            R = [rb_v[pl.ds(c * N + basen + off, L)] for c in range(9)]
            b = [rb_v[pl.ds((9 + c) * N + basen + off, L)] for c in range(3)]
            acc = jnp.zeros((L,), F32)
            for m in range(M):
                q = plsc.load_gather(neigh_v, [laneM + m])
                cxq = plsc.load_gather(cs_v, [q])
                cyq = plsc.load_gather(cs_v, [q + N])
                czq = plsc.load_gather(cs_v, [q + 2 * N])
                sxq = plsc.load_gather(cs_v, [q + 3 * N])
                syq = plsc.load_gather(cs_v, [q + 4 * N])
                szq = plsc.load_gather(cs_v, [q + 5 * N])
                dx = b[0] - cxq + (R[0] * sxq + R[1] * syq + R[2] * szq)
                dy = b[1] - cyq + (R[3] * sxq + R[4] * syq + R[5] * szq)
                dz = b[2] - czq + (R[6] * sxq + R[7] * syq + R[8] * szq)
                acc = acc + (dx * dx + dy * dy + dz * dz)
            acc = jnp.where(n_glob >= wid * NPT, acc, 0.0)
            return tot + acc

        tot = lax.fori_loop(0, NGA, arap_body, jnp.zeros((L,), F32))
        part_v[...] = tot
        pltpu.sync_copy(part_v, part_hbm.at[pl.ds(wid * L, L)])

        for ci in range(NCH):
            slot = ci % 2
            hi, hw = handles[slot]
            hi.wait()
            hw.wait()
            if ci + 1 < NCH:
                handles[(ci + 1) % 2] = fire((ci + 1) % 2, ci + 1)
            idx_v = idxb[slot]
            w_v = wb[slot]

            def body(g, carry, _ci=ci, _idx=idx_v, _w=w_v):
                llane = iota + g * L
                lane3 = (llane + _ci * CH) * 3
                laneK = llane * K
                vx = plsc.load_gather(vert_v, [lane3])
                vy = plsc.load_gather(vert_v, [lane3 + 1])
                vz = plsc.load_gather(vert_v, [lane3 + 2])
                acc = [jnp.zeros((L,), F32) for _ in range(12)]
                for kk in range(K):
                    j = plsc.load_gather(_idx, [laneK + kk])
                    w = plsc.load_gather(_w, [laneK + kk])
                    for c in range(12):
                        acc[c] = acc[c] + w * plsc.load_gather(rb_v, [j + c * N])
                ox = acc[0] * vx + acc[1] * vy + acc[2] * vz + acc[9]
                oy = acc[3] * vx + acc[4] * vy + acc[5] * vz + acc[10]
                oz = acc[6] * vx + acc[7] * vy + acc[8] * vz + acc[11]
                plsc.store_scatter(out_v, [lane3], ox)
                plsc.store_scatter(out_v, [lane3 + 1], oy)
                plsc.store_scatter(out_v, [lane3 + 2], oz)
                return carry

            lax.fori_loop(0, CG, body, jnp.int32(0))

        pltpu.sync_copy(out_v, out_hbm.at[pl.ds(base * 3, VPT * 3)])

    return k(vert_flat, inf_flat, w_flat, rb_flat, cs_flat, neigh_flat)


def kernel(vertices, opt_d_rotations, opt_d_translations, nodes_idx,
           influence_nodes_idx, weights, one_ring_neigh):
    V, K = weights.shape
    N = nodes_idx.shape[0]
    M = one_ring_neigh.shape[1]
    vert_flat = vertices.astype(F32).reshape(-1)
    nodes_idx = nodes_idx.astype(I32)
    inf_flat = influence_nodes_idx.astype(I32).reshape(-1)
    neigh_flat = one_ring_neigh.astype(I32).reshape(-1)
    w_flat = weights.astype(F32).reshape(-1)
    rot_t = opt_d_rotations[0].T.astype(F32)
    t_t = opt_d_translations[0].T.astype(F32)

    nodes_flat = _nodes_gather(vert_flat, nodes_idx)
    rb, cs = _prep(rot_t, t_t, nodes_flat.reshape(3, N))
    warped, parts = _warp_arap(
        vert_flat, inf_flat, w_flat, rb.reshape(-1), cs.reshape(-1),
        neigh_flat, V, K, N, M,
    )
    arap_loss = parts.sum() / jnp.float32(N)
    return (warped.reshape(V, 3)[None], arap_loss)


# speedup vs baseline: 2.7828x; 2.7828x over previous
import functools

import jax
import jax.numpy as jnp
from jax import lax
from jax.experimental import pallas as pl
from jax.experimental.pallas import tpu as pltpu
from jax.experimental.pallas import tpu_sc as plsc

F32 = jnp.float32
I32 = jnp.int32

NC = 2
NS = 16
L = 16
NW = NC * NS

_MESH = dict(core_axis_name="c", subcore_axis_name="s")
_NO_LAYOUT = dict(
    compiler_params=pltpu.CompilerParams(needs_layout_passes=False)
)


def _wid():
    return lax.axis_index("s") * NC + lax.axis_index("c")


def _nodes_gather(vert_pm, nodes_idx, V):
    N = nodes_idx.shape[0]
    NPT = -(-N // NW)
    NPT = -(-NPT // L) * L

    @functools.partial(
        pl.kernel,
        mesh=plsc.VectorSubcoreMesh(**_MESH),
        **_NO_LAYOUT,
        out_type=jax.ShapeDtypeStruct((3 * N,), F32),
        scratch_types=[
            pltpu.VMEM((NPT,), I32),
            pltpu.VMEM((3 * NPT,), I32),
            pltpu.VMEM((3 * NPT,), F32),
            pltpu.SemaphoreType.DMA,
        ],
    )
    def k(vert_hbm, idx_hbm, out_hbm, idx_v, idx3_v, rows_v, sem):
        base = jnp.minimum(_wid() * NPT, N - NPT)
        pltpu.sync_copy(idx_hbm.at[pl.ds(base, NPT)], idx_v)
        iota = lax.iota(I32, L)
        for g in range(NPT // L):
            j = idx_v[pl.ds(g * L, L)]
            p = iota + g * L
            for c in range(3):
                plsc.store_scatter(idx3_v, [p + c * NPT], j + c * V)
        pltpu.async_copy(vert_hbm.at[idx3_v], rows_v, sem).wait()
        for c in range(3):
            pltpu.sync_copy(
                rows_v.at[pl.ds(c * NPT, NPT)],
                out_hbm.at[pl.ds(c * N + base, NPT)],
            )

    return k(vert_pm, nodes_idx)


def _prep(rot_t, t_t, nodes_t):
    N = rot_t.shape[1]

    def body(rot_ref, t_ref, nt_ref, rb_ref, cs_ref):
        eps = jnp.float32(1e-8)
        rx = rot_ref[0:1, :]
        ry = rot_ref[1:2, :]
        rz = rot_ref[2:3, :]
        ax = rx + eps
        ay = ry + eps
        az = rz + eps
        angle = jnp.sqrt(ax * ax + ay * ay + az * az)
        inv = 1.0 / angle
        ux = rx * inv
        uy = ry * inv
        uz = rz * inv
        sn = jnp.sin(angle)
        one_c = 1.0 - jnp.cos(angle)
        xx = ux * ux
        yy = uy * uy
        zz = uz * uz
        xy = ux * uy
        xz = ux * uz
        yz = uy * uz
        r00 = 1.0 + one_c * (-zz - yy)
        r01 = sn * (-uz) + one_c * xy
        r02 = sn * uy + one_c * xz
        r10 = sn * uz + one_c * xy
        r11 = 1.0 + one_c * (-zz - xx)
        r12 = sn * (-ux) + one_c * yz
        r20 = sn * (-uy) + one_c * xz
        r21 = sn * ux + one_c * yz
        r22 = 1.0 + one_c * (-yy - xx)
        nx = nt_ref[0:1, :]
        ny = nt_ref[1:2, :]
        nz = nt_ref[2:3, :]
        tx = t_ref[0:1, :]
        ty = t_ref[1:2, :]
        tz = t_ref[2:3, :]
        bx = nx + tx - (r00 * nx + r01 * ny + r02 * nz)
        by = ny + ty - (r10 * nx + r11 * ny + r12 * nz)
        bz = nz + tz - (r20 * nx + r21 * ny + r22 * nz)
        for i, row in enumerate(
            [r00, r01, r02, r10, r11, r12, r20, r21, r22, bx, by, bz]
        ):
            rb_ref[i : i + 1, :] = row
        for i, row in enumerate([nx + tx, ny + ty, nz + tz, nx, ny, nz]):
            cs_ref[i : i + 1, :] = row

    return pl.pallas_call(
        body,
        out_shape=[
            jax.ShapeDtypeStruct((12, N), F32),
            jax.ShapeDtypeStruct((6, N), F32),
        ],
    )(rot_t, t_t, nodes_t)


def _warp_arap(vert_pm, inf_t, w_t, rb_flat, cs_flat, neigh_flat,
               V, K, N, M):
    VPT = 1568
    NG = VPT // L
    NGH = NG // 2
    VH = VPT // 2
    NPT = -(-N // NW)
    NPT = -(-NPT // L) * L
    NGA = NPT // L

    @functools.partial(
        pl.kernel,
        mesh=plsc.VectorSubcoreMesh(**_MESH),
        **_NO_LAYOUT,
        out_type=[
            jax.ShapeDtypeStruct((3 * V,), F32),
            jax.ShapeDtypeStruct((NW * L,), F32),
        ],
        scratch_types=[
            pltpu.VMEM((12 * N,), F32),
            pltpu.VMEM((6 * N,), F32),
            pltpu.VMEM((M * NPT,), I32),
            pltpu.VMEM((K * VPT,), I32),
            pltpu.VMEM((K * VH,), F32),
            pltpu.VMEM((3 * VPT,), F32),
            pltpu.VMEM((3 * VPT,), F32),
            pltpu.VMEM((L,), F32),
            pltpu.SemaphoreType.DMA,
            pltpu.SemaphoreType.DMA,
        ],
    )
    def k(vert_hbm, idx_hbm, w_hbm, rb_hbm, cs_hbm, neigh_hbm,
          out_hbm, part_hbm,
          rb_v, cs_v, neigh_v, idx_v, w_v, vert_v, out_v, part_v,
          sem_a, sem_b):
        wid = _wid()
        base = jnp.minimum(wid * VPT, V - VPT)
        basen = jnp.minimum(wid * NPT, N - NPT)
        iota = lax.iota(I32, L)

        arap_handles = [
            pltpu.async_copy(rb_hbm, rb_v, sem_b),
            pltpu.async_copy(cs_hbm, cs_v, sem_b),
            pltpu.async_copy(
                neigh_hbm.at[pl.ds(basen * M, NPT * M)], neigh_v, sem_b
            ),
        ]
        warp_handles = [
            pltpu.async_copy(
                idx_hbm.at[pl.ds(kk * V + base, VPT)],
                idx_v.at[pl.ds(kk * VPT, VPT)],
                sem_a,
            )
            for kk in range(K)
        ] + [
            pltpu.async_copy(
                vert_hbm.at[pl.ds(c * V + base, VPT)],
                vert_v.at[pl.ds(c * VPT, VPT)],
                sem_a,
            )
            for c in range(3)
        ] + [
            pltpu.async_copy(
                w_hbm.at[pl.ds(kk * V + base, VH)],
                w_v.at[pl.ds(kk * VH, VH)],
                sem_a,
            )
            for kk in range(K)
        ]
        for h in arap_handles:
            h.wait()

        def arap_body(g, tot):
            off = g * L
            laneM = (iota + off) * M
            n_glob = basen + off + iota
            R = [rb_v[pl.ds(c * N + basen + off, L)] for c in range(9)]
            b = [rb_v[pl.ds((9 + c) * N + basen + off, L)] for c in range(3)]
            acc = jnp.zeros((L,), F32)
            for m in range(M):
                q = plsc.load_gather(neigh_v, [laneM + m])
                cxq = plsc.load_gather(cs_v, [q])
                cyq = plsc.load_gather(cs_v, [q + N])
                czq = plsc.load_gather(cs_v, [q + 2 * N])
                sxq = plsc.load_gather(cs_v, [q + 3 * N])
                syq = plsc.load_gather(cs_v, [q + 4 * N])
                szq = plsc.load_gather(cs_v, [q + 5 * N])
                dx = b[0] - cxq + (R[0] * sxq + R[1] * syq + R[2] * szq)
                dy = b[1] - cyq + (R[3] * sxq + R[4] * syq + R[5] * szq)
                dz = b[2] - czq + (R[6] * sxq + R[7] * syq + R[8] * szq)
                acc = acc + (dx * dx + dy * dy + dz * dz)
            acc = jnp.where(n_glob >= wid * NPT, acc, 0.0)
            return tot + acc

        tot = lax.fori_loop(0, NGA, arap_body, jnp.zeros((L,), F32))
        part_v[...] = tot
        pltpu.sync_copy(part_v, part_hbm.at[pl.ds(wid * L, L)])

        for h in warp_handles:
            h.wait()

        for half in range(2):
            if half == 1:
                w2_handles = [
                    pltpu.async_copy(
                        w_hbm.at[pl.ds(kk * V + base + VH, VH)],
                        w_v.at[pl.ds(kk * VH, VH)],
                        sem_a,
                    )
                    for kk in range(K)
                ]
                for h in w2_handles:
                    h.wait()

            def body(g, carry, _half=half):
                lane = (_half * NGH + g) * L
                lh = g * L
                vx = vert_v[pl.ds(lane, L)]
                vy = vert_v[pl.ds(VPT + lane, L)]
                vz = vert_v[pl.ds(2 * VPT + lane, L)]
                acc = [jnp.zeros((L,), F32) for _ in range(12)]
                for kk in range(K):
                    j = idx_v[pl.ds(kk * VPT + lane, L)]
                    w = w_v[pl.ds(kk * VH + lh, L)]
                    for c in range(12):
                        acc[c] = acc[c] + w * plsc.load_gather(
                            rb_v, [j + c * N]
                        )
                ox = acc[0] * vx + acc[1] * vy + acc[2] * vz + acc[9]
                oy = acc[3] * vx + acc[4] * vy + acc[5] * vz + acc[10]
                oz = acc[6] * vx + acc[7] * vy + acc[8] * vz + acc[11]
                out_v[pl.ds(lane, L)] = ox
                out_v[pl.ds(VPT + lane, L)] = oy
                out_v[pl.ds(2 * VPT + lane, L)] = oz
                return carry

            lax.fori_loop(0, NGH, body, jnp.int32(0))

        for c in range(3):
            pltpu.sync_copy(
                out_v.at[pl.ds(c * VPT, VPT)],
                out_hbm.at[pl.ds(c * V + base, VPT)],
            )

    return k(vert_pm, inf_t, w_t, rb_flat, cs_flat, neigh_flat)


def kernel(vertices, opt_d_rotations, opt_d_translations, nodes_idx,
           influence_nodes_idx, weights, one_ring_neigh):
    V, K = weights.shape
    N = nodes_idx.shape[0]
    M = one_ring_neigh.shape[1]
    vert_pm = vertices.astype(F32).T.reshape(-1)
    nodes_idx = nodes_idx.astype(I32)
    inf_t = influence_nodes_idx.astype(I32).T.reshape(-1)
    w_t = weights.astype(F32).T.reshape(-1)
    neigh_flat = one_ring_neigh.astype(I32).reshape(-1)
    rot_t = opt_d_rotations[0].T.astype(F32)
    t_t = opt_d_translations[0].T.astype(F32)

    nodes_flat = _nodes_gather(vert_pm, nodes_idx, V)
    rb, cs = _prep(rot_t, t_t, nodes_flat.reshape(3, N))
    out_pm, parts = _warp_arap(
        vert_pm, inf_t, w_t, rb.reshape(-1), cs.reshape(-1), neigh_flat,
        V, K, N, M,
    )
    arap_loss = parts.sum() / jnp.float32(N)
    warped = out_pm.reshape(3, V).T[None]
    return (warped, arap_loss)


# speedup vs baseline: 3.0714x; 1.1037x over previous
import functools

import jax
import jax.numpy as jnp
from jax import lax
from jax.experimental import pallas as pl
from jax.experimental.pallas import tpu as pltpu
from jax.experimental.pallas import tpu_sc as plsc

F32 = jnp.float32
I32 = jnp.int32

NC = 2
NS = 16
L = 16
NW = NC * NS

_MESH = dict(core_axis_name="c", subcore_axis_name="s")
_NO_LAYOUT = dict(
    compiler_params=pltpu.CompilerParams(needs_layout_passes=False)
)


def _wid():
    return lax.axis_index("s") * NC + lax.axis_index("c")


def _nodes_gather(vert_pm, nodes_idx, V):
    N = nodes_idx.shape[0]
    NPT = -(-N // NW)
    NPT = -(-NPT // L) * L

    @functools.partial(
        pl.kernel,
        mesh=plsc.VectorSubcoreMesh(**_MESH),
        **_NO_LAYOUT,
        out_type=jax.ShapeDtypeStruct((3 * N,), F32),
        scratch_types=[
            pltpu.VMEM((NPT,), I32),
            pltpu.VMEM((3 * NPT,), I32),
            pltpu.VMEM((3 * NPT,), F32),
            pltpu.SemaphoreType.DMA,
        ],
    )
    def k(vert_hbm, idx_hbm, out_hbm, idx_v, idx3_v, rows_v, sem):
        base = jnp.minimum(_wid() * NPT, N - NPT)
        pltpu.sync_copy(idx_hbm.at[pl.ds(base, NPT)], idx_v)
        iota = lax.iota(I32, L)
        for g in range(NPT // L):
            j = idx_v[pl.ds(g * L, L)]
            p = iota + g * L
            for c in range(3):
                plsc.store_scatter(idx3_v, [p + c * NPT], j + c * V)
        pltpu.async_copy(vert_hbm.at[idx3_v], rows_v, sem).wait()
        for c in range(3):
            pltpu.sync_copy(
                rows_v.at[pl.ds(c * NPT, NPT)],
                out_hbm.at[pl.ds(c * N + base, NPT)],
            )

    return k(vert_pm, nodes_idx)


def _prep(rot_t, t_t, nodes_flat):
    N = rot_t.shape[1]

    def body(rot_ref, t_ref, nt_ref, rb_ref, cs_ref):
        eps = jnp.float32(1e-8)
        rx = rot_ref[0:1, :]
        ry = rot_ref[1:2, :]
        rz = rot_ref[2:3, :]
        ax = rx + eps
        ay = ry + eps
        az = rz + eps
        angle = jnp.sqrt(ax * ax + ay * ay + az * az)
        inv = 1.0 / angle
        ux = rx * inv
        uy = ry * inv
        uz = rz * inv
        sn = jnp.sin(angle)
        one_c = 1.0 - jnp.cos(angle)
        xx = ux * ux
        yy = uy * uy
        zz = uz * uz
        xy = ux * uy
        xz = ux * uz
        yz = uy * uz
        r00 = 1.0 + one_c * (-zz - yy)
        r01 = sn * (-uz) + one_c * xy
        r02 = sn * uy + one_c * xz
        r10 = sn * uz + one_c * xy
        r11 = 1.0 + one_c * (-zz - xx)
        r12 = sn * (-ux) + one_c * yz
        r20 = sn * (-uy) + one_c * xz
        r21 = sn * ux + one_c * yz
        r22 = 1.0 + one_c * (-yy - xx)
        nx = nt_ref[pl.ds(0, N)].reshape(1, N)
        ny = nt_ref[pl.ds(N, N)].reshape(1, N)
        nz = nt_ref[pl.ds(2 * N, N)].reshape(1, N)
        tx = t_ref[0:1, :]
        ty = t_ref[1:2, :]
        tz = t_ref[2:3, :]
        bx = nx + tx - (r00 * nx + r01 * ny + r02 * nz)
        by = ny + ty - (r10 * nx + r11 * ny + r12 * nz)
        bz = nz + tz - (r20 * nx + r21 * ny + r22 * nz)
        for i, row in enumerate(
            [r00, r01, r02, r10, r11, r12, r20, r21, r22, bx, by, bz]
        ):
            rb_ref[pl.ds(i * N, N)] = row.reshape(N)
        for i, row in enumerate([nx + tx, ny + ty, nz + tz, nx, ny, nz]):
            cs_ref[pl.ds(i * N, N)] = row.reshape(N)

    return pl.pallas_call(
        body,
        out_shape=[
            jax.ShapeDtypeStruct((12 * N,), F32),
            jax.ShapeDtypeStruct((6 * N,), F32),
        ],
    )(rot_t, t_t, nodes_flat)


def _warp_arap(vert_pm, inf_t, w_t, rb_flat, cs_flat, neigh_flat,
               V, K, N, M):
    VPT = 1568
    NG = VPT // L
    NGH = NG // 2
    VH = VPT // 2
    NPT = -(-N // NW)
    NPT = -(-NPT // L) * L
    NGA = NPT // L

    @functools.partial(
        pl.kernel,
        mesh=plsc.VectorSubcoreMesh(**_MESH),
        **_NO_LAYOUT,
        out_type=[
            jax.ShapeDtypeStruct((3 * V,), F32),
            jax.ShapeDtypeStruct((NW * L,), F32),
        ],
        scratch_types=[
            pltpu.VMEM((12 * N,), F32),
            pltpu.VMEM((6 * N,), F32),
            pltpu.VMEM((M * NPT,), I32),
            pltpu.VMEM((K * VPT,), I32),
            pltpu.VMEM((K * VH,), F32),
            pltpu.VMEM((3 * VPT,), F32),
            pltpu.VMEM((3 * VPT,), F32),
            pltpu.VMEM((L,), F32),
            pltpu.VMEM((12 * NPT,), F32),
            pltpu.SemaphoreType.DMA,
            pltpu.SemaphoreType.DMA,
        ],
    )
    def k(vert_hbm, idx_hbm, w_hbm, rb_hbm, cs_hbm, neigh_hbm,
          out_hbm, part_hbm,
          rb_v, cs_v, neigh_v, idx_v, w_v, vert_v, out_v, part_v, rbs_v,
          sem_a, sem_b):
        wid = _wid()
        base = jnp.minimum(wid * VPT, V - VPT)
        basen = jnp.minimum(wid * NPT, N - NPT)
        iota = lax.iota(I32, L)

        arap_handles = [
            pltpu.async_copy(cs_hbm, cs_v, sem_b),
            pltpu.async_copy(
                neigh_hbm.at[pl.ds(basen * M, NPT * M)], neigh_v, sem_b
            ),
        ] + [
            pltpu.async_copy(
                rb_hbm.at[pl.ds(c * N + basen, NPT)],
                rbs_v.at[pl.ds(c * NPT, NPT)],
                sem_b,
            )
            for c in range(12)
        ]
        for h in arap_handles:
            h.wait()
        warp_handles = [
            pltpu.async_copy(rb_hbm, rb_v, sem_a),
        ] + [
            pltpu.async_copy(
                idx_hbm.at[pl.ds(kk * V + base, VPT)],
                idx_v.at[pl.ds(kk * VPT, VPT)],
                sem_a,
            )
            for kk in range(K)
        ] + [
            pltpu.async_copy(
                vert_hbm.at[pl.ds(c * V + base, VPT)],
                vert_v.at[pl.ds(c * VPT, VPT)],
                sem_a,
            )
            for c in range(3)
        ] + [
            pltpu.async_copy(
                w_hbm.at[pl.ds(kk * V + base, VH)],
                w_v.at[pl.ds(kk * VH, VH)],
                sem_a,
            )
            for kk in range(K)
        ]

        def arap_body(g, tot):
            off = g * L
            laneM = (iota + off) * M
            n_glob = basen + off + iota
            R = [rbs_v[pl.ds(c * NPT + off, L)] for c in range(9)]
            b = [rbs_v[pl.ds((9 + c) * NPT + off, L)] for c in range(3)]
            acc = jnp.zeros((L,), F32)
            for m in range(M):
                q = plsc.load_gather(neigh_v, [laneM + m])
                cxq = plsc.load_gather(cs_v, [q])
                cyq = plsc.load_gather(cs_v, [q + N])
                czq = plsc.load_gather(cs_v, [q + 2 * N])
                sxq = plsc.load_gather(cs_v, [q + 3 * N])
                syq = plsc.load_gather(cs_v, [q + 4 * N])
                szq = plsc.load_gather(cs_v, [q + 5 * N])
                dx = b[0] - cxq + (R[0] * sxq + R[1] * syq + R[2] * szq)
                dy = b[1] - cyq + (R[3] * sxq + R[4] * syq + R[5] * szq)
                dz = b[2] - czq + (R[6] * sxq + R[7] * syq + R[8] * szq)
                acc = acc + (dx * dx + dy * dy + dz * dz)
            acc = jnp.where(n_glob >= wid * NPT, acc, 0.0)
            return tot + acc

        tot = lax.fori_loop(0, NGA, arap_body, jnp.zeros((L,), F32))
        part_v[...] = tot
        pltpu.sync_copy(part_v, part_hbm.at[pl.ds(wid * L, L)])

        for h in warp_handles:
            h.wait()

        for half in range(2):
            if half == 1:
                w2_handles = [
                    pltpu.async_copy(
                        w_hbm.at[pl.ds(kk * V + base + VH, VH)],
                        w_v.at[pl.ds(kk * VH, VH)],
                        sem_a,
                    )
                    for kk in range(K)
                ]
                for h in w2_handles:
                    h.wait()

            def group(g, _half=half):
                lane = (_half * NGH + g) * L
                lh = g * L
                vx = vert_v[pl.ds(lane, L)]
                vy = vert_v[pl.ds(VPT + lane, L)]
                vz = vert_v[pl.ds(2 * VPT + lane, L)]
                acc = [jnp.zeros((L,), F32) for _ in range(12)]
                for kk in range(K):
                    j = idx_v[pl.ds(kk * VPT + lane, L)]
                    w = w_v[pl.ds(kk * VH + lh, L)]
                    for c in range(12):
                        acc[c] = acc[c] + w * plsc.load_gather(
                            rb_v, [j + c * N]
                        )
                ox = acc[0] * vx + acc[1] * vy + acc[2] * vz + acc[9]
                oy = acc[3] * vx + acc[4] * vy + acc[5] * vz + acc[10]
                oz = acc[6] * vx + acc[7] * vy + acc[8] * vz + acc[11]
                out_v[pl.ds(lane, L)] = ox
                out_v[pl.ds(VPT + lane, L)] = oy
                out_v[pl.ds(2 * VPT + lane, L)] = oz

            def body2(g, carry):
                group(2 * g)
                group(2 * g + 1)
                return carry

            lax.fori_loop(0, NGH // 2, body2, jnp.int32(0))
            group(NGH - 1)

        for c in range(3):
            pltpu.sync_copy(
                out_v.at[pl.ds(c * VPT, VPT)],
                out_hbm.at[pl.ds(c * V + base, VPT)],
            )

    return k(vert_pm, inf_t, w_t, rb_flat, cs_flat, neigh_flat)


def kernel(vertices, opt_d_rotations, opt_d_translations, nodes_idx,
           influence_nodes_idx, weights, one_ring_neigh):
    V, K = weights.shape
    N = nodes_idx.shape[0]
    M = one_ring_neigh.shape[1]
    vert_pm = vertices.astype(F32).T.reshape(-1)
    nodes_idx = nodes_idx.astype(I32)
    inf_t = influence_nodes_idx.astype(I32).T.reshape(-1)
    w_t = weights.astype(F32).T.reshape(-1)
    neigh_flat = one_ring_neigh.astype(I32).reshape(-1)
    rot_t = opt_d_rotations[0].T.astype(F32)
    t_t = opt_d_translations[0].T.astype(F32)

    nodes_flat = _nodes_gather(vert_pm, nodes_idx, V)
    rb, cs = _prep(rot_t, t_t, nodes_flat)
    out_pm, parts = _warp_arap(
        vert_pm, inf_t, w_t, rb, cs, neigh_flat, V, K, N, M,
    )
    arap_loss = parts.sum() / jnp.float32(N)
    warped = out_pm.reshape(3, V).T[None]
    return (warped, arap_loss)
